# Initial kernel scaffold; baseline (speedup 1.0000x reference)
import jax
import jax.numpy as jnp
from jax.experimental import pallas as pl


def kernel(word_idx, embeddings):
    raise NotImplementedError("write your pallas kernel here")


# speedup vs baseline: 3.3714x; 3.3714x over previous
import functools

import jax
import jax.numpy as jnp
from jax import lax
from jax.experimental import pallas as pl
from jax.experimental.pallas import tpu as pltpu
from jax.experimental.pallas import tpu_sc as plsc

VOCAB = 100000
EMBED_DIM = 128
BATCH = 16384
FIELDS = 26

TOTAL = BATCH * FIELDS
NW = 32
PER_W = TOTAL // NW
CHUNK = 128
NCHUNK = PER_W // CHUNK
NBUF = 2


def _sc_gather(idx2d, table):
    mesh = plsc.VectorSubcoreMesh(core_axis_name="c", subcore_axis_name="s")

    @functools.partial(
        pl.kernel,
        mesh=mesh,
        out_type=jax.ShapeDtypeStruct((TOTAL, EMBED_DIM), jnp.float32),
        scratch_types=[
            pltpu.VMEM((NCHUNK, CHUNK), jnp.int32),
            *[pltpu.VMEM((CHUNK, EMBED_DIM), jnp.float32) for _ in range(NBUF)],
            *[pltpu.SemaphoreType.DMA for _ in range(NBUF)],
            *[pltpu.SemaphoreType.DMA for _ in range(NBUF)],
        ],
    )
    def k(idx_hbm, table_hbm, out_hbm, idx_v, buf0, buf1, g0, g1, o0, o1):
        bufs = (buf0, buf1)
        gsems = (g0, g1)
        osems = (o0, o1)
        wid = lax.axis_index("s") * 2 + lax.axis_index("c")
        row0 = wid * NCHUNK

        pltpu.sync_copy(idx_hbm.at[pl.ds(row0, NCHUNK)], idx_v)

        def gather_start(j, b):
            pltpu.make_async_copy(
                table_hbm.at[idx_v.at[j]], bufs[b], gsems[b]
            ).start()

        def gather_wait(b):
            pltpu.make_async_copy(
                table_hbm.at[idx_v.at[0]], bufs[b], gsems[b]
            ).wait()

        def out_start(j, b):
            pltpu.make_async_copy(
                bufs[b], out_hbm.at[pl.ds((row0 + j) * CHUNK, CHUNK)], osems[b]
            ).start()

        def out_wait(j, b):
            pltpu.make_async_copy(
                bufs[b], out_hbm.at[pl.ds((row0 + j) * CHUNK, CHUNK)], osems[b]
            ).wait()

        gather_start(0, 0)
        gather_start(1, 1)

        def step(i, _):
            j = i * NBUF
            for b in range(NBUF):
                gather_wait(b)
                out_start(j + b, b)
                out_wait(j + b, b)
                nxt = j + b + NBUF
                @pl.when(nxt < NCHUNK)
                def _():
                    gather_start(nxt, b)
            return 0

        lax.fori_loop(0, NCHUNK // NBUF, step, 0)

    return k(idx2d, table)


def kernel(word_idx, embeddings):
    idx2d = word_idx.reshape(TOTAL // CHUNK, CHUNK).astype(jnp.int32)
    out = _sc_gather(idx2d, embeddings)
    return out.reshape(BATCH, FIELDS, EMBED_DIM)


# speedup vs baseline: 3.3890x; 1.0052x over previous
import functools

import jax
import jax.numpy as jnp
from jax import lax
from jax.experimental import pallas as pl
from jax.experimental.pallas import tpu as pltpu
from jax.experimental.pallas import tpu_sc as plsc

VOCAB = 100000
EMBED_DIM = 128
BATCH = 16384
FIELDS = 26

TOTAL = BATCH * FIELDS
NW = 32
PER_W = TOTAL // NW
CHUNK = 128
NCHUNK = PER_W // CHUNK
NBUF = 4


def _sc_gather(idx2d, table):
    mesh = plsc.VectorSubcoreMesh(core_axis_name="c", subcore_axis_name="s")

    @functools.partial(
        pl.kernel,
        mesh=mesh,
        out_type=jax.ShapeDtypeStruct((TOTAL, EMBED_DIM), jnp.float32),
        scratch_types=[
            pltpu.VMEM((NCHUNK, CHUNK), jnp.int32),
            *[pltpu.VMEM((CHUNK, EMBED_DIM), jnp.float32) for _ in range(NBUF)],
            *[pltpu.SemaphoreType.DMA for _ in range(NBUF)],
            *[pltpu.SemaphoreType.DMA for _ in range(NBUF)],
        ],
    )
    def k(idx_hbm, table_hbm, out_hbm, idx_v,
          buf0, buf1, buf2, buf3, g0, g1, g2, g3, o0, o1, o2, o3):
        bufs = (buf0, buf1, buf2, buf3)
        gsems = (g0, g1, g2, g3)
        osems = (o0, o1, o2, o3)
        wid = lax.axis_index("s") * 2 + lax.axis_index("c")
        row0 = wid * NCHUNK

        pltpu.sync_copy(idx_hbm.at[pl.ds(row0, NCHUNK)], idx_v)

        def gather_start(j, b):
            pltpu.make_async_copy(
                table_hbm.at[idx_v.at[j]], bufs[b], gsems[b]
            ).start()

        def gather_wait(b):
            pltpu.make_async_copy(
                table_hbm.at[idx_v.at[0]], bufs[b], gsems[b]
            ).wait()

        def out_start(j, b):
            pltpu.make_async_copy(
                bufs[b], out_hbm.at[pl.ds((row0 + j) * CHUNK, CHUNK)], osems[b]
            ).start()

        def out_wait(b):
            pltpu.make_async_copy(
                bufs[b], out_hbm.at[pl.ds(row0 * CHUNK, CHUNK)], osems[b]
            ).wait()

        gather_start(0, 0)
        gather_start(1, 1)

        def step(i, _):
            c0 = i * NBUF
            for b in range(NBUF):
                c = c0 + b
                b2 = (b + 2) % NBUF

                @pl.when(c >= 2)
                def _():
                    out_wait(b2)

                @pl.when(c + 2 < NCHUNK)
                def _():
                    gather_start(c + 2, b2)

                gather_wait(b)
                out_start(c, b)
            return 0

        lax.fori_loop(0, NCHUNK // NBUF, step, 0)

        out_wait((NCHUNK - 2) % NBUF)
        out_wait((NCHUNK - 1) % NBUF)

    return k(idx2d, table)


def kernel(word_idx, embeddings):
    idx2d = word_idx.reshape(TOTAL // CHUNK, CHUNK).astype(jnp.int32)
    out = _sc_gather(idx2d, embeddings)
    return out.reshape(BATCH, FIELDS, EMBED_DIM)


# speedup vs baseline: 5.7140x; 1.6860x over previous
import functools

import jax
import jax.numpy as jnp
from jax import lax
from jax.experimental import pallas as pl
from jax.experimental.pallas import tpu as pltpu
from jax.experimental.pallas import tpu_sc as plsc

VOCAB = 100000
EMBED_DIM = 128
BATCH = 16384
FIELDS = 26

TOTAL = BATCH * FIELDS
NW = 32
BPC = 4
CHUNK = BPC * FIELDS
B_PER_W = BATCH // NW
NCHUNK = B_PER_W // BPC
NBUF = 4


def _sc_gather(idx2d, table):
    mesh = plsc.VectorSubcoreMesh(core_axis_name="c", subcore_axis_name="s")

    @functools.partial(
        pl.kernel,
        mesh=mesh,
        out_type=jax.ShapeDtypeStruct((BATCH, FIELDS, EMBED_DIM), jnp.float32),
        scratch_types=[
            pltpu.VMEM((NCHUNK, CHUNK), jnp.int32),
            *[pltpu.VMEM((CHUNK, EMBED_DIM), jnp.float32) for _ in range(NBUF)],
            *[pltpu.SemaphoreType.DMA for _ in range(NBUF)],
            *[pltpu.SemaphoreType.DMA for _ in range(NBUF)],
        ],
    )
    def k(idx_hbm, table_hbm, out_hbm, idx_v,
          buf0, buf1, buf2, buf3, g0, g1, g2, g3, o0, o1, o2, o3):
        bufs = (buf0, buf1, buf2, buf3)
        gsems = (g0, g1, g2, g3)
        osems = (o0, o1, o2, o3)
        wid = lax.axis_index("s") * 2 + lax.axis_index("c")
        row0 = wid * NCHUNK
        b0 = wid * B_PER_W

        pltpu.sync_copy(idx_hbm.at[pl.ds(row0, NCHUNK)], idx_v)

        def gather_start(j, b):
            pltpu.make_async_copy(
                table_hbm.at[idx_v.at[j]], bufs[b], gsems[b]
            ).start()

        def gather_wait(b):
            pltpu.make_async_copy(
                table_hbm.at[idx_v.at[0]], bufs[b], gsems[b]
            ).wait()

        def out_start(j, b):
            for i in range(BPC):
                pltpu.make_async_copy(
                    bufs[b].at[pl.ds(i * FIELDS, FIELDS)],
                    out_hbm.at[b0 + j * BPC + i],
                    osems[b],
                ).start()

        def out_wait(b):
            for _ in range(BPC):
                pltpu.make_async_copy(
                    bufs[b].at[pl.ds(0, FIELDS)],
                    out_hbm.at[b0],
                    osems[b],
                ).wait()

        gather_start(0, 0)
        gather_start(1, 1)

        def step(i, _):
            c0 = i * NBUF
            for b in range(NBUF):
                c = c0 + b
                b2 = (b + 2) % NBUF

                @pl.when(c >= 2)
                def _():
                    out_wait(b2)

                @pl.when(c + 2 < NCHUNK)
                def _():
                    gather_start(c + 2, b2)

                gather_wait(b)
                out_start(c, b)
            return 0

        lax.fori_loop(0, NCHUNK // NBUF, step, 0)

        out_wait((NCHUNK - 2) % NBUF)
        out_wait((NCHUNK - 1) % NBUF)

    return k(idx2d, table)


def kernel(word_idx, embeddings):
    idx2d = word_idx.reshape(TOTAL // CHUNK, CHUNK).astype(jnp.int32)
    return _sc_gather(idx2d, embeddings)
